# baseline (speedup 1.0000x reference)
import functools

import jax
import jax.numpy as jnp
from jax import lax
from jax.experimental import pallas as pl
from jax.experimental.pallas import tpu as pltpu
from jax.experimental.pallas import tpu_sc as plsc

N_NODES = 10000
NPAD = 10240
LANES = 16
CHUNK = 128
NC, NS = 2, 16
NW = NC * NS
ROWS_PER_TILE = NPAD // NS
ZCHUNKS = ROWS_PER_TILE // CHUNK
DEG_W = 16


def _sc_mesh():
    return plsc.VectorSubcoreMesh(core_axis_name="c", subcore_axis_name="s")


_SC_PARAMS = pltpu.CompilerParams(use_tc_tiling_on_sc=False)


def _zero_vmem(ref, rows, width):
    z = jnp.zeros((LANES,), jnp.float32)
    per_row = width // LANES

    def body(i, _):
        r = i // per_row
        q = i % per_row
        ref[r, pl.ds(q * LANES, LANES)] = z
        return _

    lax.fori_loop(0, rows * per_row, body, None)


def _sc_degree_hist(edge3, n_chunks):
    base = n_chunks // NW
    extra = n_chunks - base * NW

    @functools.partial(
        pl.kernel,
        out_type=jax.ShapeDtypeStruct((NC, NPAD, DEG_W), jnp.float32),
        mesh=_sc_mesh(),
        compiler_params=_SC_PARAMS,
        scratch_types=[
            pltpu.VMEM((base, CHUNK), jnp.int32),
            pltpu.VMEM((1, CHUNK), jnp.int32),
            pltpu.VMEM((CHUNK, DEG_W), jnp.float32),
            pltpu.VMEM((CHUNK, DEG_W), jnp.float32),
            pltpu.VMEM_SHARED((NPAD, DEG_W), jnp.float32),
            pltpu.SemaphoreType.DMA,
        ],
    )
    def k(edge_hbm, out_hbm, didx, dtail, ones, zbuf, acc, ssem):
        c = lax.axis_index("c")
        s = lax.axis_index("s")
        w = s * NC + c
        one = jnp.full((LANES,), 1.0, jnp.float32)

        def fill(i, _):
            ones[i, pl.ds(0, LANES)] = one
            return _

        lax.fori_loop(0, CHUNK, fill, None)
        _zero_vmem(zbuf, CHUNK, DEG_W)
        pltpu.sync_copy(edge_hbm.at[1, pl.ds(w * base, base)], didx)

        def zacc(i, _):
            pltpu.sync_copy(zbuf, acc.at[pl.ds(s * ROWS_PER_TILE + i * CHUNK, CHUNK)])
            return _

        lax.fori_loop(0, ZCHUNKS, zacc, None)
        plsc.subcore_barrier()

        HLAG = 4

        def body(j, _):
            pltpu.async_copy(ones, acc.at[didx.at[j]], ssem, add=True)

            @pl.when(j >= HLAG)
            def _retire():
                pltpu.make_async_copy(ones, acc.at[didx.at[j - HLAG]], ssem).wait()

            return _

        lax.fori_loop(0, base, body, None)
        for t in range(HLAG):
            j = base - HLAG + t
            pltpu.make_async_copy(ones, acc.at[didx.at[j]], ssem).wait()
        if extra:
            @pl.when(w < extra)
            def _tail():
                pltpu.sync_copy(edge_hbm.at[1, pl.ds(base * NW + w, 1)], dtail)
                pltpu.sync_copy(ones, acc.at[dtail.at[0]], add=True)
        plsc.subcore_barrier()

        def dump(i, _):
            off = s * ROWS_PER_TILE + i * CHUNK
            pltpu.sync_copy(acc.at[pl.ds(off, CHUNK)], out_hbm.at[c, pl.ds(off, CHUNK)])
            return _

        lax.fori_loop(0, ZCHUNKS, dump, None)

    return k(edge3)


def _sc_gather_scatter(table, edge3, n_chunks, feat):
    base = n_chunks // NW
    extra = n_chunks - base * NW
    half = (base + 1) // 2
    DEPTH, LAG = (3, 1) if feat > 32 else (6, 3)
    ZROWS = 64

    @functools.partial(
        pl.kernel,
        out_type=jax.ShapeDtypeStruct((NC, NPAD, feat), jnp.float32),
        mesh=_sc_mesh(),
        compiler_params=_SC_PARAMS,
        scratch_types=[
            pltpu.VMEM((half, CHUNK), jnp.int32),
            pltpu.VMEM((half, CHUNK), jnp.int32),
            pltpu.VMEM((2, CHUNK), jnp.int32),
            pltpu.VMEM((DEPTH, CHUNK, feat), jnp.float32),
            pltpu.VMEM((ZROWS, feat), jnp.float32),
            pltpu.VMEM_SHARED((NPAD, feat), jnp.float32),
            pltpu.VMEM_SHARED((NPAD, feat), jnp.float32),
            pltpu.SemaphoreType.DMA((DEPTH,)),
            pltpu.SemaphoreType.DMA((DEPTH,)),
        ],
    )
    def k(table_hbm, edge_hbm, out_hbm, sidx, didx, tidx, rows, zbuf,
          tspm, acc, gsem, ssem):
        c = lax.axis_index("c")
        s = lax.axis_index("s")
        w = s * NC + c
        _zero_vmem(zbuf, ZROWS, feat)
        rbase = s * ROWS_PER_TILE
        pltpu.sync_copy(table_hbm.at[pl.ds(rbase, ROWS_PER_TILE)],
                        tspm.at[pl.ds(rbase, ROWS_PER_TILE)])

        def zacc(i, _):
            pltpu.sync_copy(zbuf, acc.at[pl.ds(rbase + i * ZROWS, ZROWS)])
            return _

        lax.fori_loop(0, ROWS_PER_TILE // ZROWS, zacc, None)
        plsc.subcore_barrier()

        for h in range(2):
            lo = h * half
            cnt = min(base, lo + half) - lo
            pltpu.sync_copy(edge_hbm.at[0, pl.ds(w * base + lo, cnt)],
                            sidx.at[pl.ds(0, cnt)])
            pltpu.sync_copy(edge_hbm.at[1, pl.ds(w * base + lo, cnt)],
                            didx.at[pl.ds(0, cnt)])

            for b in range(DEPTH - LAG):
                pltpu.async_copy(tspm.at[sidx.at[b]], rows.at[b], gsem.at[b])

            def body(j, _):
                p = lax.rem(j, DEPTH)
                pltpu.make_async_copy(tspm.at[sidx.at[j]], rows.at[p],
                                      gsem.at[p]).wait()
                pltpu.async_copy(rows.at[p], acc.at[didx.at[j]], ssem.at[p],
                                 add=True)

                @pl.when(j >= LAG)
                def _retire():
                    q = lax.rem(j - LAG, DEPTH)
                    pltpu.make_async_copy(rows.at[q], acc.at[didx.at[j - LAG]],
                                          ssem.at[q]).wait()

                @pl.when(j + DEPTH - LAG < cnt)
                def _prefetch():
                    q = lax.rem(j + DEPTH - LAG, DEPTH)
                    pltpu.async_copy(tspm.at[sidx.at[j + DEPTH - LAG]],
                                     rows.at[q], gsem.at[q])

                return _

            lax.fori_loop(0, cnt, body, None)
            for t in range(LAG):
                j = cnt - LAG + t
                pltpu.make_async_copy(rows.at[j % DEPTH], acc.at[didx.at[j]],
                                      ssem.at[j % DEPTH]).wait()

        if extra:
            @pl.when(w < extra)
            def _tail():
                pltpu.sync_copy(edge_hbm.at[0, pl.ds(base * NW + w, 1)],
                                tidx.at[pl.ds(0, 1)])
                pltpu.sync_copy(edge_hbm.at[1, pl.ds(base * NW + w, 1)],
                                tidx.at[pl.ds(1, 1)])
                pltpu.sync_copy(tspm.at[tidx.at[0]], rows.at[0])
                pltpu.sync_copy(rows.at[0], acc.at[tidx.at[1]], add=True)

        plsc.subcore_barrier()

        def dump(i, _):
            off = rbase + i * CHUNK
            pltpu.sync_copy(acc.at[pl.ds(off, CHUNK)], out_hbm.at[c, pl.ds(off, CHUNK)])
            return _

        lax.fori_loop(0, ZCHUNKS, dump, None)

    return k(table, edge3)


_BLK = 1000


def _tc_matmul(x, W):
    n, kdim = x.shape
    f = W.shape[1]

    def body(xr, wr, outr):
        outr[...] = jnp.dot(xr[...], wr[...], preferred_element_type=jnp.float32)

    return pl.pallas_call(
        body,
        grid=(n // _BLK,),
        in_specs=[
            pl.BlockSpec((_BLK, kdim), lambda i: (i, 0)),
            pl.BlockSpec((kdim, f), lambda i: (0, 0)),
        ],
        out_specs=pl.BlockSpec((_BLK, f), lambda i: (i, 0)),
        out_shape=jax.ShapeDtypeStruct((NPAD, f), jnp.float32),
    )(x, W)


def _tc_prep(deg_h, xl, n):
    f = xl.shape[1]

    def body(dh, xr, xlp, dinv):
        deg = dh[0, :, :1] + dh[1, :, :1] + 1.0
        di = lax.rsqrt(deg)
        dinv[...] = jnp.broadcast_to(di, (
            di.shape[0], LANES))
        xlp[...] = di * xr[...]

    return pl.pallas_call(
        body,
        grid=(n // _BLK,),
        in_specs=[
            pl.BlockSpec((NC, _BLK, DEG_W), lambda i: (0, i, 0)),
            pl.BlockSpec((_BLK, f), lambda i: (i, 0)),
        ],
        out_specs=[
            pl.BlockSpec((_BLK, f), lambda i: (i, 0)),
            pl.BlockSpec((_BLK, LANES), lambda i: (i, 0)),
        ],
        out_shape=[
            jax.ShapeDtypeStruct((NPAD, f), jnp.float32),
            jax.ShapeDtypeStruct((n, LANES), jnp.float32),
        ],
    )(deg_h, xl)


def _tc_mid(agg, xlp, dinv, b1, W2, n):
    f = xlp.shape[1]
    f2 = W2.shape[1]

    def body(ar, xr, dr, br, wr, outr):
        di = dr[:, :1]
        h = jnp.maximum(di * (ar[0] + ar[1] + xr[...]) + br[...], 0.0)
        outr[...] = di * jnp.dot(h, wr[...], preferred_element_type=jnp.float32)

    return pl.pallas_call(
        body,
        grid=(n // _BLK,),
        in_specs=[
            pl.BlockSpec((NC, _BLK, f), lambda i: (0, i, 0)),
            pl.BlockSpec((_BLK, f), lambda i: (i, 0)),
            pl.BlockSpec((_BLK, LANES), lambda i: (i, 0)),
            pl.BlockSpec((1, f), lambda i: (0, 0)),
            pl.BlockSpec((f, f2), lambda i: (0, 0)),
        ],
        out_specs=pl.BlockSpec((_BLK, f2), lambda i: (i, 0)),
        out_shape=jax.ShapeDtypeStruct((NPAD, f2), jnp.float32),
    )(agg, xlp, dinv, b1, W2)


def _tc_final(agg, xlp, dinv, b2, Wfc, bfc, n):
    f = xlp.shape[1]
    ncls = Wfc.shape[1]
    grid = n // _BLK

    def body(ar, xr, dr, br, wr, bfr, outr, accr):
        i = pl.program_id(0)
        h = jnp.maximum(dr[:, :1] * (ar[0] + ar[1] + xr[...]) + br[...], 0.0)
        ps = jnp.sum(h, axis=0, keepdims=True)

        @pl.when(i == 0)
        def _():
            accr[...] = jnp.zeros_like(accr)

        accr[...] += ps

        @pl.when(i == grid - 1)
        def _():
            g = accr[...] * (1.0 / n)
            outr[...] = jnp.dot(g, wr[...], preferred_element_type=jnp.float32) + bfr[...]

    return pl.pallas_call(
        body,
        grid=(grid,),
        in_specs=[
            pl.BlockSpec((NC, _BLK, f), lambda i: (0, i, 0)),
            pl.BlockSpec((_BLK, f), lambda i: (i, 0)),
            pl.BlockSpec((_BLK, LANES), lambda i: (i, 0)),
            pl.BlockSpec((1, f), lambda i: (0, 0)),
            pl.BlockSpec((f, ncls), lambda i: (0, 0)),
            pl.BlockSpec((1, ncls), lambda i: (0, 0)),
        ],
        out_specs=pl.BlockSpec((1, ncls), lambda i: (0, 0)),
        out_shape=jax.ShapeDtypeStruct((1, ncls), jnp.float32),
        scratch_shapes=[pltpu.VMEM((1, f), jnp.float32)],
    )(agg, xlp, dinv, b2, Wfc, bfc)


def kernel(x, edge_index, W1, b1, W2, b2, Wfc, bfc):
    n = x.shape[0]
    e = edge_index.shape[1]
    ei = edge_index.astype(jnp.int32)
    if e % CHUNK:
        pad = CHUNK - e % CHUNK
        ei = jnp.concatenate([ei, jnp.zeros((2, pad), jnp.int32)], axis=1)
        ei = ei.at[1, e:].set(n)
    n_chunks = ei.shape[1] // CHUNK
    edge3 = ei.reshape(2, n_chunks, CHUNK)

    deg_h = _sc_degree_hist(edge3, n_chunks)
    xl = _tc_matmul(x, W1)
    xlp, dinv = _tc_prep(deg_h, xl, n)

    agg1 = _sc_gather_scatter(xlp, edge3, n_chunks, W1.shape[1])
    xl2p = _tc_mid(agg1, xlp, dinv, b1.reshape(1, -1), W2, n)

    agg2 = _sc_gather_scatter(xl2p, edge3, n_chunks, W2.shape[1])
    return _tc_final(agg2, xl2p, dinv, b2.reshape(1, -1),
                     Wfc, bfc.reshape(1, -1), n)


# speedup vs baseline: 1.0008x; 1.0008x over previous
import functools

import jax
import jax.numpy as jnp
from jax import lax
from jax.experimental import pallas as pl
from jax.experimental.pallas import tpu as pltpu
from jax.experimental.pallas import tpu_sc as plsc

N_NODES = 10000
NPAD = 10240
LANES = 16
CHUNK = 128
NC, NS = 2, 16
NW = NC * NS
ROWS_PER_TILE = NPAD // NS
ZCHUNKS = ROWS_PER_TILE // CHUNK
DEG_W = 16


def _sc_mesh():
    return plsc.VectorSubcoreMesh(core_axis_name="c", subcore_axis_name="s")


_SC_PARAMS = pltpu.CompilerParams(use_tc_tiling_on_sc=False)


def _zero_vmem(ref, rows, width):
    z = jnp.zeros((LANES,), jnp.float32)
    per_row = width // LANES

    def body(i, _):
        r = i // per_row
        q = i % per_row
        ref[r, pl.ds(q * LANES, LANES)] = z
        return _

    lax.fori_loop(0, rows * per_row, body, None)


def _sc_degree_hist(edge3, n_chunks):
    base = n_chunks // NW
    extra = n_chunks - base * NW

    @functools.partial(
        pl.kernel,
        out_type=jax.ShapeDtypeStruct((NC, NPAD, DEG_W), jnp.float32),
        mesh=_sc_mesh(),
        compiler_params=_SC_PARAMS,
        scratch_types=[
            pltpu.VMEM((base, CHUNK), jnp.int32),
            pltpu.VMEM((1, CHUNK), jnp.int32),
            pltpu.VMEM((CHUNK, DEG_W), jnp.float32),
            pltpu.VMEM((CHUNK, DEG_W), jnp.float32),
            pltpu.VMEM_SHARED((NPAD, DEG_W), jnp.float32),
            pltpu.SemaphoreType.DMA,
        ],
    )
    def k(edge_hbm, out_hbm, didx, dtail, ones, zbuf, acc, ssem):
        c = lax.axis_index("c")
        s = lax.axis_index("s")
        w = s * NC + c
        one = jnp.full((LANES,), 1.0, jnp.float32)

        def fill(i, _):
            ones[i, pl.ds(0, LANES)] = one
            return _

        lax.fori_loop(0, CHUNK, fill, None)
        _zero_vmem(zbuf, CHUNK, DEG_W)
        pltpu.sync_copy(edge_hbm.at[1, pl.ds(w * base, base)], didx)

        def zacc(i, _):
            pltpu.sync_copy(zbuf, acc.at[pl.ds(s * ROWS_PER_TILE + i * CHUNK, CHUNK)])
            return _

        lax.fori_loop(0, ZCHUNKS, zacc, None)
        plsc.subcore_barrier()

        HLAG = 4

        def body(j, _):
            pltpu.async_copy(ones, acc.at[didx.at[j]], ssem, add=True)

            @pl.when(j >= HLAG)
            def _retire():
                pltpu.make_async_copy(ones, acc.at[didx.at[j - HLAG]], ssem).wait()

            return _

        lax.fori_loop(0, base, body, None)
        for t in range(HLAG):
            j = base - HLAG + t
            pltpu.make_async_copy(ones, acc.at[didx.at[j]], ssem).wait()
        if extra:
            @pl.when(w < extra)
            def _tail():
                pltpu.sync_copy(edge_hbm.at[1, pl.ds(base * NW + w, 1)], dtail)
                pltpu.sync_copy(ones, acc.at[dtail.at[0]], add=True)
        plsc.subcore_barrier()

        def dump(i, _):
            off = s * ROWS_PER_TILE + i * CHUNK
            pltpu.sync_copy(acc.at[pl.ds(off, CHUNK)], out_hbm.at[c, pl.ds(off, CHUNK)])
            return _

        lax.fori_loop(0, ZCHUNKS, dump, None)

    return k(edge3)


def _sc_gather_scatter(table, edge3, n_chunks, feat):
    base = n_chunks // NW
    extra = n_chunks - base * NW
    half = (base + 1) // 2
    DEPTH, LAG = (3, 1) if feat > 32 else (6, 3)
    ZROWS = 64

    @functools.partial(
        pl.kernel,
        out_type=jax.ShapeDtypeStruct((NC, NPAD, feat), jnp.float32),
        mesh=_sc_mesh(),
        compiler_params=_SC_PARAMS,
        scratch_types=[
            pltpu.VMEM((half, CHUNK), jnp.int32),
            pltpu.VMEM((half, CHUNK), jnp.int32),
            pltpu.VMEM((2, CHUNK), jnp.int32),
            pltpu.VMEM((DEPTH, CHUNK, feat), jnp.float32),
            pltpu.VMEM((ZROWS, feat), jnp.float32),
            pltpu.VMEM_SHARED((NPAD, feat), jnp.float32),
            pltpu.VMEM_SHARED((NPAD, feat), jnp.float32),
            pltpu.SemaphoreType.DMA((DEPTH,)),
            pltpu.SemaphoreType.DMA((DEPTH,)),
        ],
    )
    def k(table_hbm, edge_hbm, out_hbm, sidx, didx, tidx, rows, zbuf,
          tspm, acc, gsem, ssem):
        c = lax.axis_index("c")
        s = lax.axis_index("s")
        w = s * NC + c
        _zero_vmem(zbuf, ZROWS, feat)
        rbase = s * ROWS_PER_TILE
        pltpu.sync_copy(table_hbm.at[pl.ds(rbase, ROWS_PER_TILE)],
                        tspm.at[pl.ds(rbase, ROWS_PER_TILE)])

        def zacc(i, _):
            pltpu.sync_copy(zbuf, acc.at[pl.ds(rbase + i * ZROWS, ZROWS)])
            return _

        lax.fori_loop(0, ROWS_PER_TILE // ZROWS, zacc, None)
        plsc.subcore_barrier()

        for h in range(2):
            lo = h * half
            cnt = min(base, lo + half) - lo
            pltpu.sync_copy(edge_hbm.at[0, pl.ds(w * base + lo, cnt)],
                            sidx.at[pl.ds(0, cnt)])
            pltpu.sync_copy(edge_hbm.at[1, pl.ds(w * base + lo, cnt)],
                            didx.at[pl.ds(0, cnt)])

            for b in range(DEPTH - LAG):
                pltpu.async_copy(tspm.at[sidx.at[b]], rows.at[b], gsem.at[b])

            def body(j, _):
                p = lax.rem(j, DEPTH)
                pltpu.make_async_copy(tspm.at[sidx.at[j]], rows.at[p],
                                      gsem.at[p]).wait()
                pltpu.async_copy(rows.at[p], acc.at[didx.at[j]], ssem.at[p],
                                 add=True)

                @pl.when(j >= LAG)
                def _retire():
                    q = lax.rem(j - LAG, DEPTH)
                    pltpu.make_async_copy(rows.at[q], acc.at[didx.at[j - LAG]],
                                          ssem.at[q]).wait()

                @pl.when(j + DEPTH - LAG < cnt)
                def _prefetch():
                    q = lax.rem(j + DEPTH - LAG, DEPTH)
                    pltpu.async_copy(tspm.at[sidx.at[j + DEPTH - LAG]],
                                     rows.at[q], gsem.at[q])

                return _

            lax.fori_loop(0, cnt, body, None)
            for t in range(LAG):
                j = cnt - LAG + t
                pltpu.make_async_copy(rows.at[j % DEPTH], acc.at[didx.at[j]],
                                      ssem.at[j % DEPTH]).wait()

        if extra:
            @pl.when(w < extra)
            def _tail():
                pltpu.sync_copy(edge_hbm.at[0, pl.ds(base * NW + w, 1)],
                                tidx.at[pl.ds(0, 1)])
                pltpu.sync_copy(edge_hbm.at[1, pl.ds(base * NW + w, 1)],
                                tidx.at[pl.ds(1, 1)])
                pltpu.sync_copy(tspm.at[tidx.at[0]], rows.at[0])
                pltpu.sync_copy(rows.at[0], acc.at[tidx.at[1]], add=True)

        plsc.subcore_barrier()

        def dump(i, _):
            off = rbase + i * CHUNK
            pltpu.sync_copy(acc.at[pl.ds(off, CHUNK)], out_hbm.at[c, pl.ds(off, CHUNK)])
            return _

        lax.fori_loop(0, ZCHUNKS, dump, None)

    return k(table, edge3)


_BLK = 1000


def _tc_matmul(x, W):
    n, kdim = x.shape
    f = W.shape[1]

    def body(xr, wr, outr):
        outr[...] = jnp.dot(xr[...], wr[...], preferred_element_type=jnp.float32)

    return pl.pallas_call(
        body,
        grid=(n // _BLK,),
        in_specs=[
            pl.BlockSpec((_BLK, kdim), lambda i: (i, 0)),
            pl.BlockSpec((kdim, f), lambda i: (0, 0)),
        ],
        out_specs=pl.BlockSpec((_BLK, f), lambda i: (i, 0)),
        out_shape=jax.ShapeDtypeStruct((NPAD, f), jnp.float32),
    )(x, W)


def _tc_prep(deg_h, xl, n):
    f = xl.shape[1]

    def body(dh, xr, xlp, dinv):
        deg = dh[0, :, :1] + dh[1, :, :1] + 1.0
        di = lax.rsqrt(deg)
        dinv[...] = jnp.broadcast_to(di, (
            di.shape[0], LANES))
        xlp[...] = di * xr[...]

    return pl.pallas_call(
        body,
        grid=(n // _BLK,),
        in_specs=[
            pl.BlockSpec((NC, _BLK, DEG_W), lambda i: (0, i, 0)),
            pl.BlockSpec((_BLK, f), lambda i: (i, 0)),
        ],
        out_specs=[
            pl.BlockSpec((_BLK, f), lambda i: (i, 0)),
            pl.BlockSpec((_BLK, LANES), lambda i: (i, 0)),
        ],
        out_shape=[
            jax.ShapeDtypeStruct((NPAD, f), jnp.float32),
            jax.ShapeDtypeStruct((n, LANES), jnp.float32),
        ],
    )(deg_h, xl)


def _tc_mid(agg, xlp, dinv, b1, W2, n, f):
    f2 = W2.shape[1]

    def body(ar, xr, dr, br, wr, outr):
        di = dr[:, :1]
        h = jnp.maximum(di * (ar[0] + ar[1] + xr[...]) + br[...], 0.0)
        outr[...] = di * jnp.dot(h, wr[...], preferred_element_type=jnp.float32)

    return pl.pallas_call(
        body,
        grid=(n // _BLK,),
        in_specs=[
            pl.BlockSpec((NC, _BLK, f), lambda i: (0, i, 0)),
            pl.BlockSpec((_BLK, f), lambda i: (i, 0)),
            pl.BlockSpec((_BLK, LANES), lambda i: (i, 0)),
            pl.BlockSpec((1, f), lambda i: (0, 0)),
            pl.BlockSpec((f, f2), lambda i: (0, 0)),
        ],
        out_specs=pl.BlockSpec((_BLK, f2), lambda i: (i, 0)),
        out_shape=jax.ShapeDtypeStruct((NPAD, f2), jnp.float32),
    )(agg, xlp, dinv, b1, W2)


def _tc_final(agg, xlp, dinv, b2, Wfc, bfc, n, f):
    ncls = Wfc.shape[1]
    grid = n // _BLK

    def body(ar, xr, dr, br, wr, bfr, outr, accr):
        i = pl.program_id(0)
        h = jnp.maximum(dr[:, :1] * (ar[0] + ar[1] + xr[...]) + br[...], 0.0)
        ps = jnp.sum(h, axis=0, keepdims=True)

        @pl.when(i == 0)
        def _():
            accr[...] = jnp.zeros_like(accr)

        accr[...] += ps

        @pl.when(i == grid - 1)
        def _():
            g = accr[...] * (1.0 / n)
            outr[...] = jnp.dot(g, wr[...], preferred_element_type=jnp.float32) + bfr[...]

    return pl.pallas_call(
        body,
        grid=(grid,),
        in_specs=[
            pl.BlockSpec((NC, _BLK, f), lambda i: (0, i, 0)),
            pl.BlockSpec((_BLK, f), lambda i: (i, 0)),
            pl.BlockSpec((_BLK, LANES), lambda i: (i, 0)),
            pl.BlockSpec((1, f), lambda i: (0, 0)),
            pl.BlockSpec((f, ncls), lambda i: (0, 0)),
            pl.BlockSpec((1, ncls), lambda i: (0, 0)),
        ],
        out_specs=pl.BlockSpec((1, ncls), lambda i: (0, 0)),
        out_shape=jax.ShapeDtypeStruct((1, ncls), jnp.float32),
        scratch_shapes=[pltpu.VMEM((1, f), jnp.float32)],
    )(agg, xlp, dinv, b2, Wfc, bfc)


def kernel(x, edge_index, W1, b1, W2, b2, Wfc, bfc):
    n = x.shape[0]
    e = edge_index.shape[1]
    ei = edge_index.astype(jnp.int32)
    if e % CHUNK:
        pad = CHUNK - e % CHUNK
        ei = jnp.concatenate([ei, jnp.zeros((2, pad), jnp.int32)], axis=1)
        ei = ei.at[1, e:].set(n)
    n_chunks = ei.shape[1] // CHUNK
    edge3 = ei.reshape(2, n_chunks, CHUNK)

    deg_h = _sc_degree_hist(edge3, n_chunks)
    xl = _tc_matmul(x, W1)
    xlp, dinv = _tc_prep(deg_h, xl, n)

    agg1 = _sc_gather_scatter(xlp, edge3, n_chunks, W1.shape[1])
    xl2p = _tc_mid(agg1, xlp, dinv, b1.reshape(1, -1), W2, n, W1.shape[1])

    agg2 = _sc_gather_scatter(xl2p, edge3, n_chunks, W2.shape[1])
    return _tc_final(agg2, xl2p, dinv, b2.reshape(1, -1),
                     Wfc, bfc.reshape(1, -1), n, W2.shape[1])


# speedup vs baseline: 1.0312x; 1.0304x over previous
import functools

import jax
import jax.numpy as jnp
from jax import lax
from jax.experimental import pallas as pl
from jax.experimental.pallas import tpu as pltpu
from jax.experimental.pallas import tpu_sc as plsc

N_NODES = 10000
NPAD = 10240
LANES = 16
CHUNK = 128
NC, NS = 2, 16
NW = NC * NS
ROWS_PER_TILE = NPAD // NS
ZCHUNKS = ROWS_PER_TILE // CHUNK
DEG_W = 16


def _sc_mesh():
    return plsc.VectorSubcoreMesh(core_axis_name="c", subcore_axis_name="s")


_SC_PARAMS = pltpu.CompilerParams(use_tc_tiling_on_sc=False)


def _zero_vmem(ref, rows, width):
    z = jnp.zeros((LANES,), jnp.float32)
    per_row = width // LANES

    def body(i, _):
        r = i // per_row
        q = i % per_row
        ref[r, pl.ds(q * LANES, LANES)] = z
        return _

    lax.fori_loop(0, rows * per_row, body, None)


def _sc_degree_hist(edge3, n_chunks):
    base = n_chunks // NW
    extra = n_chunks - base * NW

    @functools.partial(
        pl.kernel,
        out_type=jax.ShapeDtypeStruct((NC, NPAD, DEG_W), jnp.float32),
        mesh=_sc_mesh(),
        compiler_params=_SC_PARAMS,
        scratch_types=[
            pltpu.VMEM((base, CHUNK), jnp.int32),
            pltpu.VMEM((1, CHUNK), jnp.int32),
            pltpu.VMEM((CHUNK, DEG_W), jnp.float32),
            pltpu.VMEM((CHUNK, DEG_W), jnp.float32),
            pltpu.VMEM_SHARED((NPAD, DEG_W), jnp.float32),
            pltpu.SemaphoreType.DMA,
        ],
    )
    def k(edge_hbm, out_hbm, didx, dtail, ones, zbuf, acc, ssem):
        c = lax.axis_index("c")
        s = lax.axis_index("s")
        w = s * NC + c
        one = jnp.full((LANES,), 1.0, jnp.float32)

        def fill(i, _):
            ones[i, pl.ds(0, LANES)] = one
            return _

        lax.fori_loop(0, CHUNK, fill, None)
        _zero_vmem(zbuf, CHUNK, DEG_W)
        pltpu.sync_copy(edge_hbm.at[1, pl.ds(w * base, base)], didx)

        def zacc(i, _):
            pltpu.sync_copy(zbuf, acc.at[pl.ds(s * ROWS_PER_TILE + i * CHUNK, CHUNK)])
            return _

        lax.fori_loop(0, ZCHUNKS, zacc, None)
        plsc.subcore_barrier()

        HLAG = 4

        def body(j, _):
            pltpu.async_copy(ones, acc.at[didx.at[j]], ssem, add=True)

            @pl.when(j >= HLAG)
            def _retire():
                pltpu.make_async_copy(ones, acc.at[didx.at[j - HLAG]], ssem).wait()

            return _

        lax.fori_loop(0, base, body, None)
        for t in range(HLAG):
            j = base - HLAG + t
            pltpu.make_async_copy(ones, acc.at[didx.at[j]], ssem).wait()
        if extra:
            @pl.when(w < extra)
            def _tail():
                pltpu.sync_copy(edge_hbm.at[1, pl.ds(base * NW + w, 1)], dtail)
                pltpu.sync_copy(ones, acc.at[dtail.at[0]], add=True)
        plsc.subcore_barrier()

        def dump(i, _):
            off = s * ROWS_PER_TILE + i * CHUNK
            pltpu.sync_copy(acc.at[pl.ds(off, CHUNK)], out_hbm.at[c, pl.ds(off, CHUNK)])
            return _

        lax.fori_loop(0, ZCHUNKS, dump, None)

    return k(edge3)


def _sc_gather_scatter(table, edge3, n_chunks, feat):
    base = n_chunks // NW
    extra = n_chunks - base * NW
    half = (base + 1) // 2
    DEPTH, LAG = (3, 1) if feat > 32 else (6, 3)
    ZROWS = 64

    @functools.partial(
        pl.kernel,
        out_type=jax.ShapeDtypeStruct((NC, NPAD, feat), jnp.float32),
        mesh=_sc_mesh(),
        compiler_params=_SC_PARAMS,
        scratch_types=[
            pltpu.VMEM((half, CHUNK), jnp.int32),
            pltpu.VMEM((half, CHUNK), jnp.int32),
            pltpu.VMEM((2, CHUNK), jnp.int32),
            pltpu.VMEM((DEPTH, CHUNK, feat), jnp.float32),
            pltpu.VMEM((ZROWS, feat), jnp.float32),
            pltpu.VMEM_SHARED((NPAD, feat), jnp.float32),
            pltpu.VMEM_SHARED((NPAD, feat), jnp.float32),
            pltpu.SemaphoreType.DMA((DEPTH,)),
            pltpu.SemaphoreType.DMA((DEPTH,)),
        ],
    )
    def k(table_hbm, edge_hbm, out_hbm, sidx, didx, tidx, rows, zbuf,
          tspm, acc, gsem, ssem):
        c = lax.axis_index("c")
        s = lax.axis_index("s")
        w = s * NC + c
        _zero_vmem(zbuf, ZROWS, feat)
        rbase = s * ROWS_PER_TILE
        pltpu.sync_copy(table_hbm.at[pl.ds(rbase, ROWS_PER_TILE)],
                        tspm.at[pl.ds(rbase, ROWS_PER_TILE)])

        def zacc(i, _):
            pltpu.sync_copy(zbuf, acc.at[pl.ds(rbase + i * ZROWS, ZROWS)])
            return _

        lax.fori_loop(0, ROWS_PER_TILE // ZROWS, zacc, None)
        plsc.subcore_barrier()

        for h in range(2):
            lo = h * half
            cnt = min(base, lo + half) - lo
            pltpu.sync_copy(edge_hbm.at[0, pl.ds(w * base + lo, cnt)],
                            sidx.at[pl.ds(0, cnt)])
            pltpu.sync_copy(edge_hbm.at[1, pl.ds(w * base + lo, cnt)],
                            didx.at[pl.ds(0, cnt)])

            for b in range(DEPTH - LAG):
                pltpu.async_copy(tspm.at[sidx.at[b]], rows.at[b], gsem.at[b])

            def body(j, _):
                p = lax.rem(j, DEPTH)
                pltpu.make_async_copy(tspm.at[sidx.at[j]], rows.at[p],
                                      gsem.at[p]).wait()
                pltpu.async_copy(rows.at[p], acc.at[didx.at[j]], ssem.at[p],
                                 add=True)

                @pl.when(j >= LAG)
                def _retire():
                    q = lax.rem(j - LAG, DEPTH)
                    pltpu.make_async_copy(rows.at[q], acc.at[didx.at[j - LAG]],
                                          ssem.at[q]).wait()

                @pl.when(j + DEPTH - LAG < cnt)
                def _prefetch():
                    q = lax.rem(j + DEPTH - LAG, DEPTH)
                    pltpu.async_copy(tspm.at[sidx.at[j + DEPTH - LAG]],
                                     rows.at[q], gsem.at[q])

                return _

            lax.fori_loop(0, cnt, body, None)
            for t in range(LAG):
                j = cnt - LAG + t
                pltpu.make_async_copy(rows.at[j % DEPTH], acc.at[didx.at[j]],
                                      ssem.at[j % DEPTH]).wait()

        if extra:
            @pl.when(w < extra)
            def _tail():
                pltpu.sync_copy(edge_hbm.at[0, pl.ds(base * NW + w, 1)],
                                tidx.at[pl.ds(0, 1)])
                pltpu.sync_copy(edge_hbm.at[1, pl.ds(base * NW + w, 1)],
                                tidx.at[pl.ds(1, 1)])
                pltpu.sync_copy(tspm.at[tidx.at[0]], rows.at[0])
                pltpu.sync_copy(rows.at[0], acc.at[tidx.at[1]], add=True)

        plsc.subcore_barrier()

        def dump(i, _):
            off = rbase + i * CHUNK
            pltpu.sync_copy(acc.at[pl.ds(off, CHUNK)], out_hbm.at[c, pl.ds(off, CHUNK)])
            return _

        lax.fori_loop(0, ZCHUNKS, dump, None)

    return k(table, edge3)


_BLK = 2000


def _tc_matmul(x, W):
    n, kdim = x.shape
    f = W.shape[1]

    def body(xr, wr, outr):
        outr[...] = jnp.dot(xr[...], wr[...], preferred_element_type=jnp.float32)

    return pl.pallas_call(
        body,
        grid=(n // _BLK,),
        in_specs=[
            pl.BlockSpec((_BLK, kdim), lambda i: (i, 0)),
            pl.BlockSpec((kdim, f), lambda i: (0, 0)),
        ],
        out_specs=pl.BlockSpec((_BLK, f), lambda i: (i, 0)),
        out_shape=jax.ShapeDtypeStruct((NPAD, f), jnp.float32),
    )(x, W)


def _tc_prep(deg_h, xl, n):
    f = xl.shape[1]

    def body(dh, xr, xlp, dinv):
        deg = dh[0, :, :1] + dh[1, :, :1] + 1.0
        di = lax.rsqrt(deg)
        dinv[...] = jnp.broadcast_to(di, (
            di.shape[0], LANES))
        xlp[...] = di * xr[...]

    return pl.pallas_call(
        body,
        grid=(n // _BLK,),
        in_specs=[
            pl.BlockSpec((NC, _BLK, DEG_W), lambda i: (0, i, 0)),
            pl.BlockSpec((_BLK, f), lambda i: (i, 0)),
        ],
        out_specs=[
            pl.BlockSpec((_BLK, f), lambda i: (i, 0)),
            pl.BlockSpec((_BLK, LANES), lambda i: (i, 0)),
        ],
        out_shape=[
            jax.ShapeDtypeStruct((NPAD, f), jnp.float32),
            jax.ShapeDtypeStruct((n, LANES), jnp.float32),
        ],
    )(deg_h, xl)


def _tc_mid(agg, xlp, dinv, b1, W2, n, f):
    f2 = W2.shape[1]

    def body(ar, xr, dr, br, wr, outr):
        di = dr[:, :1]
        h = jnp.maximum(di * (ar[0] + ar[1] + xr[...]) + br[...], 0.0)
        outr[...] = di * jnp.dot(h, wr[...], preferred_element_type=jnp.float32)

    return pl.pallas_call(
        body,
        grid=(n // _BLK,),
        in_specs=[
            pl.BlockSpec((NC, _BLK, f), lambda i: (0, i, 0)),
            pl.BlockSpec((_BLK, f), lambda i: (i, 0)),
            pl.BlockSpec((_BLK, LANES), lambda i: (i, 0)),
            pl.BlockSpec((1, f), lambda i: (0, 0)),
            pl.BlockSpec((f, f2), lambda i: (0, 0)),
        ],
        out_specs=pl.BlockSpec((_BLK, f2), lambda i: (i, 0)),
        out_shape=jax.ShapeDtypeStruct((NPAD, f2), jnp.float32),
    )(agg, xlp, dinv, b1, W2)


def _tc_final(agg, xlp, dinv, b2, Wfc, bfc, n, f):
    ncls = Wfc.shape[1]
    grid = n // _BLK

    def body(ar, xr, dr, br, wr, bfr, outr, accr):
        i = pl.program_id(0)
        h = jnp.maximum(dr[:, :1] * (ar[0] + ar[1] + xr[...]) + br[...], 0.0)
        ps = jnp.sum(h, axis=0, keepdims=True)

        @pl.when(i == 0)
        def _():
            accr[...] = jnp.zeros_like(accr)

        accr[...] += ps

        @pl.when(i == grid - 1)
        def _():
            g = accr[...] * (1.0 / n)
            outr[...] = jnp.dot(g, wr[...], preferred_element_type=jnp.float32) + bfr[...]

    return pl.pallas_call(
        body,
        grid=(grid,),
        in_specs=[
            pl.BlockSpec((NC, _BLK, f), lambda i: (0, i, 0)),
            pl.BlockSpec((_BLK, f), lambda i: (i, 0)),
            pl.BlockSpec((_BLK, LANES), lambda i: (i, 0)),
            pl.BlockSpec((1, f), lambda i: (0, 0)),
            pl.BlockSpec((f, ncls), lambda i: (0, 0)),
            pl.BlockSpec((1, ncls), lambda i: (0, 0)),
        ],
        out_specs=pl.BlockSpec((1, ncls), lambda i: (0, 0)),
        out_shape=jax.ShapeDtypeStruct((1, ncls), jnp.float32),
        scratch_shapes=[pltpu.VMEM((1, f), jnp.float32)],
    )(agg, xlp, dinv, b2, Wfc, bfc)


def kernel(x, edge_index, W1, b1, W2, b2, Wfc, bfc):
    n = x.shape[0]
    e = edge_index.shape[1]
    ei = edge_index.astype(jnp.int32)
    if e % CHUNK:
        pad = CHUNK - e % CHUNK
        ei = jnp.concatenate([ei, jnp.zeros((2, pad), jnp.int32)], axis=1)
        ei = ei.at[1, e:].set(n)
    n_chunks = ei.shape[1] // CHUNK
    edge3 = ei.reshape(2, n_chunks, CHUNK)

    deg_h = _sc_degree_hist(edge3, n_chunks)
    xl = _tc_matmul(x, W1)
    xlp, dinv = _tc_prep(deg_h, xl, n)

    agg1 = _sc_gather_scatter(xlp, edge3, n_chunks, W1.shape[1])
    xl2p = _tc_mid(agg1, xlp, dinv, b1.reshape(1, -1), W2, n, W1.shape[1])

    agg2 = _sc_gather_scatter(xl2p, edge3, n_chunks, W2.shape[1])
    return _tc_final(agg2, xl2p, dinv, b2.reshape(1, -1),
                     Wfc, bfc.reshape(1, -1), n, W2.shape[1])
